# baseline (speedup 1.0000x reference)
import functools

import jax
import jax.numpy as jnp
from jax import lax
from jax.experimental import pallas as pl
from jax.experimental.pallas import tpu as pltpu
from jax.experimental.pallas import tpu_sc as plsc

B = 1024
F = 26
VOCAB = 1000
EMB = 128
DM = 512
NG = 13
NT = 8

NW = 32
BH = B // 2
PAIRS = BH * NG
NROWS = 2 * PAIRS
RW = NROWS // NW
CR = 104
NCHUNK = RW // CR


def _sc_gather(table, allidx):
    mesh = plsc.VectorSubcoreMesh(core_axis_name="c", subcore_axis_name="s")

    @functools.partial(
        pl.kernel,
        mesh=mesh,
        out_type=jax.ShapeDtypeStruct((NROWS, EMB), jnp.float32),
        scratch_types=[
            pltpu.VMEM((CR,), jnp.int32),
            pltpu.VMEM((CR,), jnp.int32),
            pltpu.VMEM((CR, EMB), jnp.float32),
            pltpu.VMEM((CR, EMB), jnp.float32),
            pltpu.SemaphoreType.DMA,
            pltpu.SemaphoreType.DMA,
            pltpu.SemaphoreType.DMA,
            pltpu.SemaphoreType.DMA,
        ],
    )
    def k(table_h, idx_h, out_h, idx_v0, idx_v1, buf_v0, buf_v1,
          semg0, semg1, semo0, semo1):
        wid = lax.axis_index("s") * 2 + lax.axis_index("c")
        idx_v = (idx_v0, idx_v1)
        buf_v = (buf_v0, buf_v1)
        semg = (semg0, semg1)
        semo = (semo0, semo1)

        def stage(ci):
            b = ci % 2
            rbase = wid * RW + ci * CR
            pltpu.sync_copy(idx_h.at[pl.ds(rbase, CR)], idx_v[b])
            return pltpu.async_copy(table_h.at[idx_v[b]], buf_v[b], semg[b])

        gcp = stage(0)
        outcp = [None, None]
        for ci in range(NCHUNK):
            b = ci % 2
            cur = gcp
            if ci + 1 < NCHUNK:
                nb = (ci + 1) % 2
                if outcp[nb] is not None:
                    outcp[nb].wait()
                gcp = stage(ci + 1)
            cur.wait()
            rbase = wid * RW + ci * CR
            outcp[b] = pltpu.async_copy(buf_v[b], out_h.at[pl.ds(rbase, CR)],
                                        semo[b])
        if outcp[NCHUNK % 2] is not None:
            outcp[NCHUNK % 2].wait()
        outcp[(NCHUNK - 1) % 2].wait()

    return k(table, allidx)


BT = 128


def _silu(x):
    return x / (1.0 + jnp.exp(-x))


def _tc_body(x4, mf, me2, w1, b1, g1, be1, rms3, w2, b2, g2, be2, outr, hbuf):
    ss = jnp.zeros((BT, 1), jnp.float32)
    w1v = w1[...]
    miss = jnp.dot(mf[...], me2[...], preferred_element_type=jnp.float32)
    for g in range(NG):
        xg = ((x4[0, g] + x4[1, g]).astype(jnp.float32) * 0.5
              + miss[:, g * EMB:(g + 1) * EMB])
        h = jnp.dot(xg.astype(jnp.bfloat16), w1v,
                    preferred_element_type=jnp.float32)
        h = _silu(h + b1[...])
        mu = jnp.mean(h, axis=1, keepdims=True)
        hc = h - mu
        var = jnp.mean(hc * hc, axis=1, keepdims=True)
        h = hc * lax.rsqrt(var + 1e-5) * g1[...] + be1[...]
        ss = ss + jnp.sum(h * h, axis=1, keepdims=True)
        hbuf[:, pl.ds(g * DM, DM)] = (h * rms3[g]).astype(jnp.bfloat16)
    inv = lax.rsqrt(ss * (1.0 / (NG * DM)) + 1e-6)
    y = jnp.dot(hbuf[...], w2[...], preferred_element_type=jnp.float32) * inv
    y = _silu(y + b2[...])
    mu = jnp.mean(y, axis=1, keepdims=True)
    yc = y - mu
    var = jnp.mean(yc * yc, axis=1, keepdims=True)
    outr[...] = yc * lax.rsqrt(var + 1e-5) * g2[...] + be2[...]


def _tc_call(x4, mf, me2, w1, b1, g1, be1, rms3, w2bf, b2, g2, be2):
    return pl.pallas_call(
        _tc_body,
        grid=(BH // BT,),
        in_specs=[
            pl.BlockSpec((2, NG, BT, EMB), lambda i: (0, 0, i, 0)),
            pl.BlockSpec((BT, F), lambda i: (i, 0)),
            pl.BlockSpec((F, NG * EMB), lambda i: (0, 0)),
            pl.BlockSpec((EMB, DM), lambda i: (0, 0)),
            pl.BlockSpec((1, DM), lambda i: (0, 0)),
            pl.BlockSpec((1, DM), lambda i: (0, 0)),
            pl.BlockSpec((1, DM), lambda i: (0, 0)),
            pl.BlockSpec((NG, 1, DM), lambda i: (0, 0, 0)),
            pl.BlockSpec((NG * DM, NT * DM), lambda i: (0, 0)),
            pl.BlockSpec((1, NT * DM), lambda i: (0, 0)),
            pl.BlockSpec((1, NT * DM), lambda i: (0, 0)),
            pl.BlockSpec((1, NT * DM), lambda i: (0, 0)),
        ],
        out_specs=pl.BlockSpec((BT, NT * DM), lambda i: (i, 0)),
        out_shape=jax.ShapeDtypeStruct((BH, NT * DM), jnp.float32),
        scratch_shapes=[pltpu.VMEM((BT, NG * DM), jnp.bfloat16)],
        compiler_params=pltpu.CompilerParams(vmem_limit_bytes=100 * 1024 * 1024),
    )(x4, mf, me2, w1, b1, g1, be1, rms3, w2bf, b2, g2, be2)


def kernel(int_feats, missing_mask, table, missing_emb, W1, b1, ln1_g, ln1_b,
           rms_s, W2, b2, ln2_g, ln2_b):
    offs = (jnp.arange(F, dtype=jnp.int32) * VOCAB)[None, :]
    idx = int_feats.astype(jnp.int32) + offs
    mf = missing_mask.astype(jnp.float32)
    onehot = (jnp.arange(NG)[None, :] == (jnp.arange(F) // 2)[:, None])
    me2 = (onehot.astype(jnp.float32)[:, :, None]
           * missing_emb[:, None, :]).reshape(F, NG * EMB) * 0.5
    w2bf = W2.astype(jnp.bfloat16)
    w1bf = W1.astype(jnp.bfloat16)
    smalls = (b1.reshape(1, DM), ln1_g.reshape(1, DM), ln1_b.reshape(1, DM),
              rms_s.reshape(NG, 1, DM))
    tails = (b2.reshape(1, NT * DM), ln2_g.reshape(1, NT * DM),
             ln2_b.reshape(1, NT * DM))

    outs = []
    for h in range(2):
        idx_h = idx[h * BH:(h + 1) * BH]
        allidx = jnp.concatenate(
            [idx_h[:, 0::2].T.reshape(-1), idx_h[:, 1::2].T.reshape(-1)])
        rows = _sc_gather(table, allidx)
        x4 = rows.reshape(2, NG, BH, EMB).astype(jnp.bfloat16)
        outs.append(_tc_call(
            x4, mf[h * BH:(h + 1) * BH], me2, w1bf, *smalls, w2bf, *tails))
    return jnp.concatenate(outs, axis=0).reshape(B, NT, DM)
